# baseline (speedup 1.0000x reference)
import functools

import jax
import jax.numpy as jnp
from jax import lax
from jax.experimental import pallas as pl
from jax.experimental.pallas import tpu as pltpu
from jax.experimental.pallas import tpu_sc as plsc

N = 10000
D = 128
E = 320000
ALPHA = 0.7
BETA = 0.3
GAMMA = 0.2

NC = 2
NS = 16
NW = NC * NS
EPT = E // NW
ROWS = 80
RPAD = ROWS * 128

_mesh = plsc.VectorSubcoreMesh(core_axis_name="c", subcore_axis_name="s")


def _scores_body(wa_ref, wfc_ref, h_ref, out_ref):
    A = jnp.dot(wa_ref[...], wfc_ref[...], preferred_element_type=jnp.float32)
    s = lax.dot_general(
        A, h_ref[...], (((1,), (1,)), ((), ())),
        preferred_element_type=jnp.float32)
    out_ref[...] = jnp.pad(s, ((0, 0), (0, RPAD - N)))


def _scores(wa, wfc, h):
    return pl.pallas_call(
        _scores_body,
        out_shape=jax.ShapeDtypeStruct((2, RPAD), jnp.float32),
    )(wa, wfc, h)


def _edge_body(s_hbm, sd_hbm, p_hbm, dpart_hbm,
               s1_t, s2_t, sd_t, p_t, zb_t, dsum, sem):
    cid = lax.axis_index("c")
    sid = lax.axis_index("s")
    wid = sid * NC + cid

    pltpu.sync_copy(s_hbm.at[0], s1_t)
    pltpu.sync_copy(s_hbm.at[1], s2_t)
    pltpu.sync_copy(sd_hbm.at[wid], sd_t)

    for c in range(128):
        zb_t[pl.ds(c * 16, 16)] = jnp.zeros((16,), jnp.float32)

    @pl.when(sid == 0)
    def _():
        for k in range(RPAD // 2048):
            pltpu.sync_copy(zb_t, dsum.at[pl.ds(k * 2048, 2048)])

    plsc.subcore_barrier()

    iota16 = lax.broadcasted_iota(jnp.int32, (16,), 0)

    def row(r, _):
        def col(c, _):
            srcv = sd_t[r, 0, pl.ds(c * 16, 16)]
            dstv = sd_t[r, 1, pl.ds(c * 16, 16)]
            s1v = plsc.load_gather(s1_t, [srcv])
            s2v = plsc.load_gather(s2_t, [dstv])
            a = s1v + s2v
            e = jnp.where(a > 0, a, GAMMA * a)
            e = jnp.minimum(e, 60.0)
            p = jnp.exp(e)
            lid = r * 128 + c * 16 + iota16
            p = jnp.where(lid < EPT, p, 0.0)
            p_t[r, 0, pl.ds(c * 16, 16)] = p
            return 0
        lax.fori_loop(0, 8, col, 0)
        return 0
    lax.fori_loop(0, ROWS, row, 0)

    def srow(r, _):
        pltpu.sync_copy(p_t.at[r, 0], dsum.at[sd_t.at[r, 1]], add=True)
        return 0
    lax.fori_loop(0, ROWS, srow, 0)

    plsc.subcore_barrier()

    pltpu.sync_copy(p_t, p_hbm.at[wid])

    @pl.when(sid == 0)
    def _():
        pltpu.sync_copy(dsum, dpart_hbm.at[cid])


@functools.partial(
    pl.kernel,
    out_type=(jax.ShapeDtypeStruct((NW, ROWS, 1, 128), jnp.float32),
              jax.ShapeDtypeStruct((NC, RPAD), jnp.float32)),
    mesh=_mesh,
    compiler_params=pltpu.CompilerParams(needs_layout_passes=False),
    scratch_types=[
        pltpu.VMEM((RPAD,), jnp.float32),
        pltpu.VMEM((RPAD,), jnp.float32),
        pltpu.VMEM((ROWS, 2, 128), jnp.int32),
        pltpu.VMEM((ROWS, 1, 128), jnp.float32),
        pltpu.VMEM((2048,), jnp.float32),
        pltpu.VMEM_SHARED((RPAD,), jnp.float32),
        pltpu.SemaphoreType.DMA,
    ],
)
def _edge_kernel(s_hbm, sd_hbm, p_hbm, dpart_hbm, *scratch):
    _edge_body(s_hbm, sd_hbm, p_hbm, dpart_hbm, *scratch)


def _dmerge_body(d_ref, out_ref):
    out_ref[...] = d_ref[0] + d_ref[1]


def _dmerge(dpart):
    return pl.pallas_call(
        _dmerge_body,
        out_shape=jax.ShapeDtypeStruct((ROWS, 128), jnp.float32),
    )(dpart.reshape(NC, ROWS, 128))


_DNUMS = lax.GatherDimensionNumbers(
    offset_dims=(), collapsed_slice_dims=(0,), start_index_map=(0,))
DH = D // 2


def _msg_body(denom_hbm, p_hbm, sd_hbm, hh_hbm, acc_hbm,
              denom_t, sd0, sd1, sd2, sd3, pr0, pr1, pr2, pr3,
              rba, rbb, hsp, acc,
              si0, si1, si2, si3, sga, sgb, ssa, ssb):
    cid = lax.axis_index("c")
    sid = lax.axis_index("s")
    wid = sid * NC + cid
    sd = (sd0, sd1, sd2, sd3)
    pr = (pr0, pr1, pr2, pr3)
    si = (si0, si1, si2, si3)
    rb = (rba, rbb)
    sg = (sga, sgb)
    ss = (ssa, ssb)
    base = sid * 640

    pltpu.sync_copy(denom_hbm, denom_t)

    def start_idx(r, s, with_p):
        pltpu.async_copy(sd_hbm.at[wid, r], sd[s], si[s])
        if with_p:
            pltpu.async_copy(p_hbm.at[wid, r], pr[s], si[s])

    def wait_idx(r, s, with_p):
        pltpu.make_async_copy(sd_hbm.at[wid, r], sd[s], si[s]).wait()
        if with_p:
            pltpu.make_async_copy(p_hbm.at[wid, r], pr[s], si[s]).wait()

    def start_gather(s, b):
        pltpu.async_copy(hsp.at[sd[s].at[0]], rb[b], sg[b])

    def wait_gather(b):
        pltpu.make_async_copy(hsp.at[sd[0].at[0]], rb[b], sg[b]).wait()

    def start_scatter(s, b):
        pltpu.async_copy(rb[b], acc.at[sd[s].at[1]], ss[b], add=True)

    def wait_scatter(b):
        pltpu.make_async_copy(rb[b], acc.at[sd[0].at[1]], ss[b]).wait()

    for half in range(2):
        with_p = True

        def zrow(i, _):
            def zcol(c, _):
                rba[i, pl.ds(c * 16, 16)] = jnp.zeros((16,), jnp.float32)
                return 0
            lax.fori_loop(0, DH // 16, zcol, 0)
            return 0
        lax.fori_loop(0, 128, zrow, 0)
        for k in range(5):
            pltpu.sync_copy(rba, acc.at[pl.ds(base + k * 128, 128)])
        for k in range(5):
            pltpu.sync_copy(hh_hbm.at[half, pl.ds(base + k * 128, 128)], rba)
            pltpu.sync_copy(rba, hsp.at[pl.ds(base + k * 128, 128)])

        plsc.subcore_barrier()

        start_idx(0, 0, with_p)
        start_idx(1, 1, with_p)
        wait_idx(0, 0, with_p)
        start_gather(0, 0)

        def quad(i, _):
            for k in range(4):
                r = i * 4 + k
                b = k % 2
                s = k

                @pl.when(r + 2 < ROWS)
                def _():
                    start_idx(r + 2, (k + 2) % 4, with_p)

                @pl.when(r + 1 < ROWS)
                def _():
                    wait_idx(r + 1, (k + 1) % 4, with_p)

                    @pl.when(r >= 1)
                    def _():
                        wait_scatter(1 - b)
                    start_gather((k + 1) % 4, 1 - b)

                wait_gather(b)

                def scale(c, _):
                    dstv = sd[s][1, pl.ds(c * 16, 16)]
                    dv = plsc.load_gather(denom_t, [dstv])
                    pv = pr[s][0, pl.ds(c * 16, 16)]
                    attnv = pv / (dv + 1e-16)
                    for j in range(16):
                        sp = lax.gather(
                            attnv, jnp.full((16, 1), j, jnp.int32),
                            _DNUMS, (1,),
                            mode=lax.GatherScatterMode.PROMISE_IN_BOUNDS)
                        row = c * 16 + j
                        for dch in range(DH // 16):
                            rb[b][row, pl.ds(dch * 16, 16)] = (
                                rb[b][row, pl.ds(dch * 16, 16)] * sp)
                    return 0
                lax.fori_loop(0, 8, scale, 0)

                start_scatter(s, b)
            return 0
        lax.fori_loop(0, ROWS // 4, quad, 0)

        wait_scatter(0)
        wait_scatter(1)

        plsc.subcore_barrier()

        for k in range(5):
            pltpu.sync_copy(acc.at[pl.ds(base + k * 128, 128)],
                            acc_hbm.at[half, cid, pl.ds(base + k * 128, 128)])


@functools.partial(
    pl.kernel,
    out_type=jax.ShapeDtypeStruct((2, NC, RPAD, DH), jnp.float32),
    mesh=_mesh,
    compiler_params=pltpu.CompilerParams(
        needs_layout_passes=False, use_tc_tiling_on_sc=False),
    scratch_types=[
        pltpu.VMEM((RPAD,), jnp.float32),
        pltpu.VMEM((2, 128), jnp.int32),
        pltpu.VMEM((2, 128), jnp.int32),
        pltpu.VMEM((2, 128), jnp.int32),
        pltpu.VMEM((2, 128), jnp.int32),
        pltpu.VMEM((1, 128), jnp.float32),
        pltpu.VMEM((1, 128), jnp.float32),
        pltpu.VMEM((1, 128), jnp.float32),
        pltpu.VMEM((1, 128), jnp.float32),
        pltpu.VMEM((128, DH), jnp.float32),
        pltpu.VMEM((128, DH), jnp.float32),
        pltpu.VMEM_SHARED((RPAD, DH), jnp.float32),
        pltpu.VMEM_SHARED((RPAD, DH), jnp.float32),
        pltpu.SemaphoreType.DMA,
        pltpu.SemaphoreType.DMA,
        pltpu.SemaphoreType.DMA,
        pltpu.SemaphoreType.DMA,
        pltpu.SemaphoreType.DMA,
        pltpu.SemaphoreType.DMA,
        pltpu.SemaphoreType.DMA,
        pltpu.SemaphoreType.DMA,
    ],
)
def _msg_kernel(denom_hbm, p_hbm, sd_hbm, hh_hbm, acc_hbm, *scratch):
    _msg_body(denom_hbm, p_hbm, sd_hbm, hh_hbm, acc_hbm, *scratch)


def _blend_body(emb_ref, acc_ref, out_ref):
    crf = jnp.concatenate(
        [acc_ref[0, 0] + acc_ref[0, 1], acc_ref[1, 0] + acc_ref[1, 1]],
        axis=-1)
    out_ref[...] = (ALPHA * emb_ref[...] + BETA * crf) / (ALPHA + BETA)


def _blend(emb, acc):
    blk = 1000
    return pl.pallas_call(
        _blend_body,
        grid=(N // blk,),
        in_specs=[pl.BlockSpec((blk, D), lambda g: (g, 0)),
                  pl.BlockSpec((2, NC, blk, DH), lambda g: (0, 0, g, 0))],
        out_specs=pl.BlockSpec((blk, D), lambda g: (g, 0)),
        out_shape=jax.ShapeDtypeStruct((N, D), jnp.float32),
    )(emb, acc)


def kernel(embedding_input, h_input, edge_index, W_fc, W_attn):
    wa = W_attn.reshape(2, D)
    s = _scores(wa, W_fc, h_input)

    src = edge_index[0].reshape(NW, EPT)
    dst = edge_index[1].reshape(NW, EPT)
    src = jnp.pad(src, ((0, 0), (0, RPAD - EPT))).reshape(NW, ROWS, 1, 128)
    dst = jnp.pad(dst, ((0, 0), (0, RPAD - EPT))).reshape(NW, ROWS, 1, 128)
    sd = jnp.concatenate([src, dst], axis=2)

    hh = jnp.stack([h_input[:, :DH], h_input[:, DH:]], axis=0)
    hh = jnp.pad(hh, ((0, 0), (0, RPAD - N), (0, 0)))

    p, dpart = _edge_kernel(s, sd)
    denom = _dmerge(dpart).reshape(RPAD)
    acc = _msg_kernel(denom, p, sd, hh)
    return _blend(embedding_input, acc)


# speedup vs baseline: 1.1075x; 1.1075x over previous
import functools

import jax
import jax.numpy as jnp
from jax import lax
from jax.experimental import pallas as pl
from jax.experimental.pallas import tpu as pltpu
from jax.experimental.pallas import tpu_sc as plsc

N = 10000
D = 128
E = 320000
ALPHA = 0.7
BETA = 0.3
GAMMA = 0.2

NC = 2
NS = 16
NW = NC * NS
EPT = E // NW
ROWS = 80
RPAD = ROWS * 128

_mesh = plsc.VectorSubcoreMesh(core_axis_name="c", subcore_axis_name="s")


def _scores_body(wa_ref, wfc_ref, h_ref, out_ref):
    A = jnp.dot(wa_ref[...], wfc_ref[...], preferred_element_type=jnp.float32)
    s = lax.dot_general(
        A, h_ref[...], (((1,), (1,)), ((), ())),
        preferred_element_type=jnp.float32)
    out_ref[...] = jnp.pad(s, ((0, 0), (0, RPAD - N)))


def _scores(wa, wfc, h):
    return pl.pallas_call(
        _scores_body,
        out_shape=jax.ShapeDtypeStruct((2, RPAD), jnp.float32),
    )(wa, wfc, h)


def _edge_body(s_hbm, sd_hbm, p_hbm, dpart_hbm,
               s1_t, s2_t, sd_t, p_t, zb_t, dsum, sem):
    cid = lax.axis_index("c")
    sid = lax.axis_index("s")
    wid = sid * NC + cid

    pltpu.sync_copy(s_hbm.at[0], s1_t)
    pltpu.sync_copy(s_hbm.at[1], s2_t)
    pltpu.sync_copy(sd_hbm.at[wid], sd_t)

    for c in range(128):
        zb_t[pl.ds(c * 16, 16)] = jnp.zeros((16,), jnp.float32)

    @pl.when(sid == 0)
    def _():
        for k in range(RPAD // 2048):
            pltpu.sync_copy(zb_t, dsum.at[pl.ds(k * 2048, 2048)])

    plsc.subcore_barrier()

    iota16 = lax.broadcasted_iota(jnp.int32, (16,), 0)

    def row(r, _):
        def col(c, _):
            srcv = sd_t[r, 0, pl.ds(c * 16, 16)]
            dstv = sd_t[r, 1, pl.ds(c * 16, 16)]
            s1v = plsc.load_gather(s1_t, [srcv])
            s2v = plsc.load_gather(s2_t, [dstv])
            a = s1v + s2v
            e = jnp.where(a > 0, a, GAMMA * a)
            e = jnp.minimum(e, 60.0)
            p = jnp.exp(e)
            lid = r * 128 + c * 16 + iota16
            p = jnp.where(lid < EPT, p, 0.0)
            p_t[r, 0, pl.ds(c * 16, 16)] = p
            return 0
        lax.fori_loop(0, 8, col, 0)
        return 0
    lax.fori_loop(0, ROWS, row, 0)

    def srow(r, _):
        pltpu.sync_copy(p_t.at[r, 0], dsum.at[sd_t.at[r, 1]], add=True)
        return 0
    lax.fori_loop(0, ROWS, srow, 0)

    plsc.subcore_barrier()

    pltpu.sync_copy(p_t, p_hbm.at[wid])

    @pl.when(sid == 0)
    def _():
        pltpu.sync_copy(dsum, dpart_hbm.at[cid])


@functools.partial(
    pl.kernel,
    out_type=(jax.ShapeDtypeStruct((NW, ROWS, 1, 128), jnp.float32),
              jax.ShapeDtypeStruct((NC, RPAD), jnp.float32)),
    mesh=_mesh,
    compiler_params=pltpu.CompilerParams(needs_layout_passes=False),
    scratch_types=[
        pltpu.VMEM((RPAD,), jnp.float32),
        pltpu.VMEM((RPAD,), jnp.float32),
        pltpu.VMEM((ROWS, 2, 128), jnp.int32),
        pltpu.VMEM((ROWS, 1, 128), jnp.float32),
        pltpu.VMEM((2048,), jnp.float32),
        pltpu.VMEM_SHARED((RPAD,), jnp.float32),
        pltpu.SemaphoreType.DMA,
    ],
)
def _edge_kernel(s_hbm, sd_hbm, p_hbm, dpart_hbm, *scratch):
    _edge_body(s_hbm, sd_hbm, p_hbm, dpart_hbm, *scratch)


def _dmerge_body(d_ref, out_ref):
    out_ref[...] = d_ref[0] + d_ref[1]


def _dmerge(dpart):
    return pl.pallas_call(
        _dmerge_body,
        out_shape=jax.ShapeDtypeStruct((ROWS, 128), jnp.float32),
    )(dpart.reshape(NC, ROWS, 128))


_DNUMS = lax.GatherDimensionNumbers(
    offset_dims=(), collapsed_slice_dims=(0,), start_index_map=(0,))
NB2 = 160
BE = 64


def _msg_body(denom_hbm, p_hbm, sd_hbm, h_hbm, acc_hbm,
              denom_t, sd0, sd1, sd2, sd3, sd4, sd5, sd6, sd7,
              pr0, pr1, pr2, pr3, pr4, pr5, pr6, pr7,
              rb0, rb1, rb2, rb3, acc,
              si0, si1, si2, si3, si4, si5, si6, si7,
              sg0, sg1, sg2, sg3, ss0, ss1, ss2, ss3):
    cid = lax.axis_index("c")
    sid = lax.axis_index("s")
    wid = sid * NC + cid
    sd = (sd0, sd1, sd2, sd3, sd4, sd5, sd6, sd7)
    pr = (pr0, pr1, pr2, pr3, pr4, pr5, pr6, pr7)
    si = (si0, si1, si2, si3, si4, si5, si6, si7)
    rb = (rb0, rb1, rb2, rb3)
    sg = (sg0, sg1, sg2, sg3)
    ss = (ss0, ss1, ss2, ss3)
    base = sid * 640

    pltpu.sync_copy(denom_hbm, denom_t)

    def zrow(i, _):
        def zcol(c, _):
            rb0[i, pl.ds(c * 16, 16)] = jnp.zeros((16,), jnp.float32)
            return 0
        lax.fori_loop(0, 8, zcol, 0)
        return 0
    lax.fori_loop(0, BE, zrow, 0)
    for k in range(10):
        pltpu.sync_copy(rb0, acc.at[pl.ds(base + k * BE, BE)])

    plsc.subcore_barrier()

    def start_idx(r, s):
        pltpu.async_copy(sd_hbm.at[wid, r], sd[s], si[s])
        pltpu.async_copy(p_hbm.at[wid, r], pr[s], si[s])

    def wait_idx(r, s):
        pltpu.make_async_copy(sd_hbm.at[wid, r], sd[s], si[s]).wait()
        pltpu.make_async_copy(p_hbm.at[wid, r], pr[s], si[s]).wait()

    def start_gather(s, b):
        pltpu.async_copy(h_hbm.at[sd[s].at[0]], rb[b], sg[b])

    def wait_gather(b):
        pltpu.make_async_copy(h_hbm.at[sd[0].at[0]], rb[b], sg[b]).wait()

    def start_scatter(s, b):
        pltpu.async_copy(rb[b], acc.at[sd[s].at[1]], ss[b], add=True)

    def wait_scatter(b):
        pltpu.make_async_copy(rb[b], acc.at[sd[0].at[1]], ss[b]).wait()

    start_idx(0, 0)
    start_idx(1, 1)
    start_idx(2, 2)
    wait_idx(0, 0)
    start_gather(0, 0)
    wait_idx(1, 1)
    start_gather(1, 1)

    def oct_(i, _):
        for k in range(8):
            r = i * 8 + k
            b = k % 4
            s = k

            @pl.when(r + 3 < NB2)
            def _():
                start_idx(r + 3, (k + 3) % 8)

            @pl.when(r + 2 < NB2)
            def _():
                @pl.when(r >= 2)
                def _():
                    wait_scatter((k + 2) % 4)
                wait_idx(r + 2, (k + 2) % 8)
                start_gather((k + 2) % 8, (k + 2) % 4)

            wait_gather(b)

            def scale(c, _):
                dstv = sd[s][1, pl.ds(c * 16, 16)]
                dv = plsc.load_gather(denom_t, [dstv])
                pv = pr[s][0, pl.ds(c * 16, 16)]
                attnv = pv / (dv + 1e-16)
                for j in range(16):
                    sp = lax.gather(
                        attnv, jnp.full((16, 1), j, jnp.int32), _DNUMS, (1,),
                        mode=lax.GatherScatterMode.PROMISE_IN_BOUNDS)
                    row = c * 16 + j
                    for dch in range(8):
                        rb[b][row, pl.ds(dch * 16, 16)] = (
                            rb[b][row, pl.ds(dch * 16, 16)] * sp)
                return 0
            lax.fori_loop(0, BE // 16, scale, 0)

            start_scatter(s, b)
        return 0
    lax.fori_loop(0, NB2 // 8, oct_, 0)

    for b in range(4):
        wait_scatter(b)

    plsc.subcore_barrier()

    for k in range(5):
        pltpu.sync_copy(acc.at[pl.ds(base + k * 128, 128)],
                        acc_hbm.at[cid, pl.ds(base + k * 128, 128)])


@functools.partial(
    pl.kernel,
    out_type=jax.ShapeDtypeStruct((NC, RPAD, D), jnp.float32),
    mesh=_mesh,
    compiler_params=pltpu.CompilerParams(needs_layout_passes=False),
    scratch_types=(
        [pltpu.VMEM((RPAD,), jnp.float32)]
        + [pltpu.VMEM((2, BE), jnp.int32)] * 8
        + [pltpu.VMEM((1, BE), jnp.float32)] * 8
        + [pltpu.VMEM((BE, D), jnp.float32)] * 4
        + [pltpu.VMEM_SHARED((RPAD, D), jnp.float32)]
        + [pltpu.SemaphoreType.DMA] * 16
    ),
)
def _msg_kernel(denom_hbm, p_hbm, sd_hbm, h_hbm, acc_hbm, *scratch):
    _msg_body(denom_hbm, p_hbm, sd_hbm, h_hbm, acc_hbm, *scratch)


def _blend_body(emb_ref, acc_ref, out_ref):
    out_ref[...] = (ALPHA * emb_ref[...]
                    + BETA * (acc_ref[0] + acc_ref[1])) / (ALPHA + BETA)


def _blend(emb, acc):
    blk = 1000
    return pl.pallas_call(
        _blend_body,
        grid=(N // blk,),
        in_specs=[pl.BlockSpec((blk, D), lambda g: (g, 0)),
                  pl.BlockSpec((NC, blk, D), lambda g: (0, g, 0))],
        out_specs=pl.BlockSpec((blk, D), lambda g: (g, 0)),
        out_shape=jax.ShapeDtypeStruct((N, D), jnp.float32),
    )(emb, acc)


def kernel(embedding_input, h_input, edge_index, W_fc, W_attn):
    wa = W_attn.reshape(2, D)
    s = _scores(wa, W_fc, h_input)

    src = edge_index[0].reshape(NW, EPT)
    dst = edge_index[1].reshape(NW, EPT)
    srcp = jnp.pad(src, ((0, 0), (0, RPAD - EPT)))
    dstp = jnp.pad(dst, ((0, 0), (0, RPAD - EPT)))
    sd = jnp.concatenate([srcp.reshape(NW, ROWS, 1, 128),
                          dstp.reshape(NW, ROWS, 1, 128)], axis=2)
    sd64 = jnp.concatenate([srcp.reshape(NW, NB2, 1, BE),
                            dstp.reshape(NW, NB2, 1, BE)], axis=2)

    p, dpart = _edge_kernel(s, sd)
    denom = _dmerge(dpart).reshape(RPAD)
    acc = _msg_kernel(denom, p.reshape(NW, NB2, 1, BE), sd64, h_input)
    return _blend(embedding_input, acc)


# speedup vs baseline: 1.2138x; 1.0959x over previous
import functools

import jax
import jax.numpy as jnp
from jax import lax
from jax.experimental import pallas as pl
from jax.experimental.pallas import tpu as pltpu
from jax.experimental.pallas import tpu_sc as plsc

N = 10000
D = 128
E = 320000
ALPHA = 0.7
BETA = 0.3
GAMMA = 0.2

NC = 2
NS = 16
NW = NC * NS
EPT = E // NW
ROWS = 80
RPAD = ROWS * 128

_mesh = plsc.VectorSubcoreMesh(core_axis_name="c", subcore_axis_name="s")


def _scores_body(wa_ref, wfc_ref, h_ref, out_ref):
    A = jnp.dot(wa_ref[...], wfc_ref[...], preferred_element_type=jnp.float32)
    s = lax.dot_general(
        A, h_ref[...], (((1,), (1,)), ((), ())),
        preferred_element_type=jnp.float32)
    out_ref[...] = jnp.pad(s, ((0, 0), (0, RPAD - N)))


def _scores(wa, wfc, h):
    return pl.pallas_call(
        _scores_body,
        out_shape=jax.ShapeDtypeStruct((2, RPAD), jnp.float32),
    )(wa, wfc, h)


def _edge_body(s_hbm, sd_hbm, p_hbm, dpart_hbm,
               s1_t, s2_t, sd_t, p_t, zb_t, dsum, sem):
    cid = lax.axis_index("c")
    sid = lax.axis_index("s")
    wid = sid * NC + cid

    pltpu.sync_copy(s_hbm.at[0], s1_t)
    pltpu.sync_copy(s_hbm.at[1], s2_t)
    pltpu.sync_copy(sd_hbm.at[wid], sd_t)

    for c in range(128):
        zb_t[pl.ds(c * 16, 16)] = jnp.zeros((16,), jnp.float32)

    @pl.when(sid == 0)
    def _():
        for k in range(RPAD // 2048):
            pltpu.sync_copy(zb_t, dsum.at[pl.ds(k * 2048, 2048)])

    plsc.subcore_barrier()

    iota16 = lax.broadcasted_iota(jnp.int32, (16,), 0)

    def row(r, _):
        def col(c, _):
            srcv = sd_t[r, 0, pl.ds(c * 16, 16)]
            dstv = sd_t[r, 1, pl.ds(c * 16, 16)]
            s1v = plsc.load_gather(s1_t, [srcv])
            s2v = plsc.load_gather(s2_t, [dstv])
            a = s1v + s2v
            e = jnp.where(a > 0, a, GAMMA * a)
            e = jnp.minimum(e, 60.0)
            p = jnp.exp(e)
            lid = r * 128 + c * 16 + iota16
            p = jnp.where(lid < EPT, p, 0.0)
            p_t[r, 0, pl.ds(c * 16, 16)] = p
            return 0
        lax.fori_loop(0, 8, col, 0)
        return 0
    lax.fori_loop(0, ROWS, row, 0)

    def srow(r, _):
        pltpu.sync_copy(p_t.at[r, 0], dsum.at[sd_t.at[r, 1]], add=True)
        return 0
    lax.fori_loop(0, ROWS, srow, 0)

    plsc.subcore_barrier()

    pltpu.sync_copy(p_t, p_hbm.at[wid])

    @pl.when(sid == 0)
    def _():
        pltpu.sync_copy(dsum, dpart_hbm.at[cid])


@functools.partial(
    pl.kernel,
    out_type=(jax.ShapeDtypeStruct((NW, ROWS, 1, 128), jnp.float32),
              jax.ShapeDtypeStruct((NC, RPAD), jnp.float32)),
    mesh=_mesh,
    compiler_params=pltpu.CompilerParams(needs_layout_passes=False),
    scratch_types=[
        pltpu.VMEM((RPAD,), jnp.float32),
        pltpu.VMEM((RPAD,), jnp.float32),
        pltpu.VMEM((ROWS, 2, 128), jnp.int32),
        pltpu.VMEM((ROWS, 1, 128), jnp.float32),
        pltpu.VMEM((2048,), jnp.float32),
        pltpu.VMEM_SHARED((RPAD,), jnp.float32),
        pltpu.SemaphoreType.DMA,
    ],
)
def _edge_kernel(s_hbm, sd_hbm, p_hbm, dpart_hbm, *scratch):
    _edge_body(s_hbm, sd_hbm, p_hbm, dpart_hbm, *scratch)


def _dmerge_body(d_ref, out_ref):
    out_ref[...] = d_ref[0] + d_ref[1]


def _dmerge(dpart):
    return pl.pallas_call(
        _dmerge_body,
        out_shape=jax.ShapeDtypeStruct((ROWS, 128), jnp.float32),
    )(dpart.reshape(NC, ROWS, 128))


_DNUMS = lax.GatherDimensionNumbers(
    offset_dims=(), collapsed_slice_dims=(0,), start_index_map=(0,))
NB2 = 160
BE = 64


def _msg_body(denom_hbm, p_hbm, sd_hbm, h_hbm, acc_hbm,
              denom_t, sd0, sd1, sd2, sd3, sd4, sd5, sd6, sd7,
              pr0, pr1, pr2, pr3, pr4, pr5, pr6, pr7,
              rb0, rb1, rb2, rb3, acc,
              si0, si1, si2, si3, si4, si5, si6, si7,
              sg0, sg1, sg2, sg3, ss0, ss1, ss2, ss3):
    cid = lax.axis_index("c")
    sid = lax.axis_index("s")
    wid = sid * NC + cid
    sd = (sd0, sd1, sd2, sd3, sd4, sd5, sd6, sd7)
    pr = (pr0, pr1, pr2, pr3, pr4, pr5, pr6, pr7)
    si = (si0, si1, si2, si3, si4, si5, si6, si7)
    rb = (rb0, rb1, rb2, rb3)
    sg = (sg0, sg1, sg2, sg3)
    ss = (ss0, ss1, ss2, ss3)
    base = sid * 640

    pltpu.sync_copy(denom_hbm, denom_t)

    def zrow(i, _):
        def zcol(c, _):
            rb0[i, pl.ds(c * 32, 32)] = jnp.zeros((32,), jnp.bfloat16)
            return 0
        lax.fori_loop(0, 4, zcol, 0)
        return 0
    lax.fori_loop(0, BE, zrow, 0)
    for k in range(10):
        pltpu.sync_copy(rb0, acc.at[pl.ds(base + k * BE, BE)])

    plsc.subcore_barrier()

    def start_idx(r, s):
        pltpu.async_copy(sd_hbm.at[wid, r], sd[s], si[s])
        pltpu.async_copy(p_hbm.at[wid, r], pr[s], si[s])

    def wait_idx(r, s):
        pltpu.make_async_copy(sd_hbm.at[wid, r], sd[s], si[s]).wait()
        pltpu.make_async_copy(p_hbm.at[wid, r], pr[s], si[s]).wait()

    def start_gather(s, b):
        pltpu.async_copy(h_hbm.at[sd[s].at[0]], rb[b], sg[b])

    def wait_gather(b):
        pltpu.make_async_copy(h_hbm.at[sd[0].at[0]], rb[b], sg[b]).wait()

    def start_scatter(s, b):
        pltpu.async_copy(rb[b], acc.at[sd[s].at[1]], ss[b], add=True)

    def wait_scatter(b):
        pltpu.make_async_copy(rb[b], acc.at[sd[0].at[1]], ss[b]).wait()

    start_idx(0, 0)
    start_idx(1, 1)
    start_idx(2, 2)
    wait_idx(0, 0)
    start_gather(0, 0)
    wait_idx(1, 1)
    start_gather(1, 1)

    def oct_(i, _):
        for k in range(8):
            r = i * 8 + k
            b = k % 4
            s = k

            @pl.when(r + 3 < NB2)
            def _():
                start_idx(r + 3, (k + 3) % 8)

            @pl.when(r + 2 < NB2)
            def _():
                @pl.when(r >= 2)
                def _():
                    wait_scatter((k + 2) % 4)
                wait_idx(r + 2, (k + 2) % 8)
                start_gather((k + 2) % 8, (k + 2) % 4)

            wait_gather(b)

            def scale(c, _):
                dstv = sd[s][1, pl.ds(c * 16, 16)]
                dv = plsc.load_gather(denom_t, [dstv])
                pv = pr[s][0, pl.ds(c * 16, 16)]
                attnv = pv / (dv + 1e-16)
                for j in range(16):
                    sp = lax.gather(
                        attnv, jnp.full((16, 1), j, jnp.int32), _DNUMS, (1,),
                        mode=lax.GatherScatterMode.PROMISE_IN_BOUNDS)
                    spb = plsc.pack(sp, sp, format=plsc.PackFormat.INTERLEAVED)
                    row = c * 16 + j
                    for dch in range(4):
                        rb[b][row, pl.ds(dch * 32, 32)] = (
                            rb[b][row, pl.ds(dch * 32, 32)] * spb)
                return 0
            lax.fori_loop(0, BE // 16, scale, 0)

            start_scatter(s, b)
        return 0
    lax.fori_loop(0, NB2 // 8, oct_, 0)

    for b in range(4):
        wait_scatter(b)

    plsc.subcore_barrier()

    for k in range(5):
        pltpu.sync_copy(acc.at[pl.ds(base + k * 128, 128)],
                        acc_hbm.at[cid, pl.ds(base + k * 128, 128)])


@functools.partial(
    pl.kernel,
    out_type=jax.ShapeDtypeStruct((NC, RPAD, D), jnp.bfloat16),
    mesh=_mesh,
    compiler_params=pltpu.CompilerParams(
        needs_layout_passes=False, use_tc_tiling_on_sc=False),
    scratch_types=(
        [pltpu.VMEM((RPAD,), jnp.float32)]
        + [pltpu.VMEM((2, BE), jnp.int32)] * 8
        + [pltpu.VMEM((1, BE), jnp.float32)] * 8
        + [pltpu.VMEM((BE, D), jnp.bfloat16)] * 4
        + [pltpu.VMEM_SHARED((RPAD, D), jnp.bfloat16)]
        + [pltpu.SemaphoreType.DMA] * 16
    ),
)
def _msg_kernel(denom_hbm, p_hbm, sd_hbm, h_hbm, acc_hbm, *scratch):
    _msg_body(denom_hbm, p_hbm, sd_hbm, h_hbm, acc_hbm, *scratch)


def _blend_body(emb_ref, acc_ref, out_ref):
    crf = (acc_ref[0].astype(jnp.float32) + acc_ref[1].astype(jnp.float32))
    out_ref[...] = (ALPHA * emb_ref[...] + BETA * crf) / (ALPHA + BETA)


def _blend(emb, acc):
    blk = 2000
    return pl.pallas_call(
        _blend_body,
        grid=(N // blk,),
        in_specs=[pl.BlockSpec((blk, D), lambda g: (g, 0)),
                  pl.BlockSpec((NC, blk, D), lambda g: (0, g, 0))],
        out_specs=pl.BlockSpec((blk, D), lambda g: (g, 0)),
        out_shape=jax.ShapeDtypeStruct((N, D), jnp.float32),
    )(emb, acc)


def kernel(embedding_input, h_input, edge_index, W_fc, W_attn):
    wa = W_attn.reshape(2, D)
    s = _scores(wa, W_fc, h_input)

    src = edge_index[0].reshape(NW, EPT)
    dst = edge_index[1].reshape(NW, EPT)
    srcp = jnp.pad(src, ((0, 0), (0, RPAD - EPT)))
    dstp = jnp.pad(dst, ((0, 0), (0, RPAD - EPT)))
    sd = jnp.concatenate([srcp.reshape(NW, ROWS, 1, 128),
                          dstp.reshape(NW, ROWS, 1, 128)], axis=2)
    sd64 = jnp.concatenate([srcp.reshape(NW, NB2, 1, BE),
                            dstp.reshape(NW, NB2, 1, BE)], axis=2)

    p, dpart = _edge_kernel(s, sd)
    denom = _dmerge(dpart).reshape(RPAD)
    acc = _msg_kernel(denom, p.reshape(NW, NB2, 1, BE), sd64,
                      h_input.astype(jnp.bfloat16))
    return _blend(embedding_input, acc)


# speedup vs baseline: 1.6886x; 1.3912x over previous
import functools

import jax
import jax.numpy as jnp
from jax import lax
from jax.experimental import pallas as pl
from jax.experimental.pallas import tpu as pltpu
from jax.experimental.pallas import tpu_sc as plsc

N = 10000
D = 128
E = 320000
ALPHA = 0.7
BETA = 0.3
GAMMA = 0.2

NC = 2
NS = 16
NW = NC * NS
EPT = E // NW
ROWS = 80
RPAD = ROWS * 128

_mesh = plsc.VectorSubcoreMesh(core_axis_name="c", subcore_axis_name="s")


def _scores_body(wa_ref, wfc_ref, h_ref, out_ref):
    A = jnp.dot(wa_ref[...], wfc_ref[...], preferred_element_type=jnp.float32)
    s = lax.dot_general(
        A, h_ref[...], (((1,), (1,)), ((), ())),
        preferred_element_type=jnp.float32)
    out_ref[...] = jnp.pad(s, ((0, 0), (0, RPAD - N)))


def _scores(wa, wfc, h):
    return pl.pallas_call(
        _scores_body,
        out_shape=jax.ShapeDtypeStruct((2, RPAD), jnp.float32),
    )(wa, wfc, h)


def _edge_body(s_hbm, sd_hbm, p_hbm, dpart_hbm,
               s1_t, s2_t, sd_t, p_t, zb_t, dsum, sem):
    cid = lax.axis_index("c")
    sid = lax.axis_index("s")
    wid = sid * NC + cid

    pltpu.sync_copy(s_hbm.at[0], s1_t)
    pltpu.sync_copy(s_hbm.at[1], s2_t)
    pltpu.sync_copy(sd_hbm.at[wid], sd_t)

    for c in range(128):
        zb_t[pl.ds(c * 16, 16)] = jnp.zeros((16,), jnp.float32)

    @pl.when(sid == 0)
    def _():
        for k in range(RPAD // 2048):
            pltpu.sync_copy(zb_t, dsum.at[pl.ds(k * 2048, 2048)])

    plsc.subcore_barrier()

    iota16 = lax.broadcasted_iota(jnp.int32, (16,), 0)

    def row(r, _):
        def col(c, _):
            srcv = sd_t[r, 0, pl.ds(c * 16, 16)]
            dstv = sd_t[r, 1, pl.ds(c * 16, 16)]
            s1v = plsc.load_gather(s1_t, [srcv])
            s2v = plsc.load_gather(s2_t, [dstv])
            a = s1v + s2v
            e = jnp.where(a > 0, a, GAMMA * a)
            e = jnp.minimum(e, 60.0)
            p = jnp.exp(e)
            lid = r * 128 + c * 16 + iota16
            p = jnp.where(lid < EPT, p, 0.0)
            p_t[r, 0, pl.ds(c * 16, 16)] = p
            return 0
        lax.fori_loop(0, 8, col, 0)
        return 0
    lax.fori_loop(0, ROWS, row, 0)

    def srow(r, _):
        pltpu.sync_copy(p_t.at[r, 0], dsum.at[sd_t.at[r, 1]], add=True)
        return 0
    lax.fori_loop(0, ROWS, srow, 0)

    plsc.subcore_barrier()

    pltpu.sync_copy(p_t, p_hbm.at[wid])

    @pl.when(sid == 0)
    def _():
        pltpu.sync_copy(dsum, dpart_hbm.at[cid])


@functools.partial(
    pl.kernel,
    out_type=(jax.ShapeDtypeStruct((NW, ROWS, 1, 128), jnp.float32),
              jax.ShapeDtypeStruct((NC, RPAD), jnp.float32)),
    mesh=_mesh,
    compiler_params=pltpu.CompilerParams(needs_layout_passes=False),
    scratch_types=[
        pltpu.VMEM((RPAD,), jnp.float32),
        pltpu.VMEM((RPAD,), jnp.float32),
        pltpu.VMEM((ROWS, 2, 128), jnp.int32),
        pltpu.VMEM((ROWS, 1, 128), jnp.float32),
        pltpu.VMEM((2048,), jnp.float32),
        pltpu.VMEM_SHARED((RPAD,), jnp.float32),
        pltpu.SemaphoreType.DMA,
    ],
)
def _edge_kernel(s_hbm, sd_hbm, p_hbm, dpart_hbm, *scratch):
    _edge_body(s_hbm, sd_hbm, p_hbm, dpart_hbm, *scratch)


def _dmerge_body(d_ref, out_ref):
    out_ref[...] = d_ref[0] + d_ref[1]


def _dmerge(dpart):
    return pl.pallas_call(
        _dmerge_body,
        out_shape=jax.ShapeDtypeStruct((ROWS, 128), jnp.float32),
    )(dpart.reshape(NC, ROWS, 128))


_DNUMS = lax.GatherDimensionNumbers(
    offset_dims=(), collapsed_slice_dims=(0,), start_index_map=(0,))
NB2 = 160
BE = 64


def _msg_body(denom_hbm, p_hbm, sd_hbm, h_hbm, acc_hbm,
              denom_t, sd0, sd1, sd2, sd3, sd4, sd5, sd6, sd7,
              pr0, pr1, pr2, pr3, pr4, pr5, pr6, pr7,
              rb0, rb1, rb2, rb3, hsp, acc,
              si0, si1, si2, si3, si4, si5, si6, si7,
              sg0, sg1, sg2, sg3, ss0, ss1, ss2, ss3):
    cid = lax.axis_index("c")
    sid = lax.axis_index("s")
    wid = sid * NC + cid
    sd = (sd0, sd1, sd2, sd3, sd4, sd5, sd6, sd7)
    pr = (pr0, pr1, pr2, pr3, pr4, pr5, pr6, pr7)
    si = (si0, si1, si2, si3, si4, si5, si6, si7)
    rb = (rb0, rb1, rb2, rb3)
    sg = (sg0, sg1, sg2, sg3)
    ss = (ss0, ss1, ss2, ss3)
    base = sid * 640

    pltpu.sync_copy(denom_hbm, denom_t)

    def zrow(i, _):
        def zcol(c, _):
            rb0[i, pl.ds(c * 32, 32)] = jnp.zeros((32,), jnp.bfloat16)
            return 0
        lax.fori_loop(0, 4, zcol, 0)
        return 0
    lax.fori_loop(0, BE, zrow, 0)
    for k in range(10):
        pltpu.sync_copy(rb0, acc.at[pl.ds(base + k * BE, BE)])

    for k in range(10):
        pltpu.sync_copy(h_hbm.at[pl.ds(base + k * BE, BE)], rb1)
        pltpu.sync_copy(rb1, hsp.at[pl.ds(base + k * BE, BE)])

    plsc.subcore_barrier()

    def start_idx(r, s):
        pltpu.async_copy(sd_hbm.at[wid, r], sd[s], si[s])
        pltpu.async_copy(p_hbm.at[wid, r], pr[s], si[s])

    def wait_idx(r, s):
        pltpu.make_async_copy(sd_hbm.at[wid, r], sd[s], si[s]).wait()
        pltpu.make_async_copy(p_hbm.at[wid, r], pr[s], si[s]).wait()

    def start_gather(s, b):
        pltpu.async_copy(hsp.at[sd[s].at[0]], rb[b], sg[b])

    def wait_gather(b):
        pltpu.make_async_copy(hsp.at[sd[0].at[0]], rb[b], sg[b]).wait()

    def start_scatter(s, b):
        pltpu.async_copy(rb[b], acc.at[sd[s].at[1]], ss[b], add=True)

    def wait_scatter(b):
        pltpu.make_async_copy(rb[b], acc.at[sd[0].at[1]], ss[b]).wait()

    start_idx(0, 0)
    start_idx(1, 1)
    start_idx(2, 2)
    wait_idx(0, 0)
    start_gather(0, 0)
    wait_idx(1, 1)
    start_gather(1, 1)

    def oct_(i, _):
        for k in range(8):
            r = i * 8 + k
            b = k % 4
            s = k

            @pl.when(r + 3 < NB2)
            def _():
                start_idx(r + 3, (k + 3) % 8)

            @pl.when(r + 2 < NB2)
            def _():
                @pl.when(r >= 2)
                def _():
                    wait_scatter((k + 2) % 4)
                wait_idx(r + 2, (k + 2) % 8)
                start_gather((k + 2) % 8, (k + 2) % 4)

            wait_gather(b)

            def scale(c, _):
                dstv = sd[s][1, pl.ds(c * 16, 16)]
                dv = plsc.load_gather(denom_t, [dstv])
                pv = pr[s][0, pl.ds(c * 16, 16)]
                attnv = pv / (dv + 1e-16)
                for j in range(16):
                    sp = lax.gather(
                        attnv, jnp.full((16, 1), j, jnp.int32), _DNUMS, (1,),
                        mode=lax.GatherScatterMode.PROMISE_IN_BOUNDS)
                    spb = plsc.pack(sp, sp, format=plsc.PackFormat.INTERLEAVED)
                    row = c * 16 + j
                    for dch in range(4):
                        rb[b][row, pl.ds(dch * 32, 32)] = (
                            rb[b][row, pl.ds(dch * 32, 32)] * spb)
                return 0
            lax.fori_loop(0, BE // 16, scale, 0)

            start_scatter(s, b)
        return 0
    lax.fori_loop(0, NB2 // 8, oct_, 0)

    for b in range(4):
        wait_scatter(b)

    plsc.subcore_barrier()

    for k in range(5):
        pltpu.sync_copy(acc.at[pl.ds(base + k * 128, 128)],
                        acc_hbm.at[cid, pl.ds(base + k * 128, 128)])


@functools.partial(
    pl.kernel,
    out_type=jax.ShapeDtypeStruct((NC, RPAD, D), jnp.bfloat16),
    mesh=_mesh,
    compiler_params=pltpu.CompilerParams(
        needs_layout_passes=False, use_tc_tiling_on_sc=False),
    scratch_types=(
        [pltpu.VMEM((RPAD,), jnp.float32)]
        + [pltpu.VMEM((2, BE), jnp.int32)] * 8
        + [pltpu.VMEM((1, BE), jnp.float32)] * 8
        + [pltpu.VMEM((BE, D), jnp.bfloat16)] * 4
        + [pltpu.VMEM_SHARED((RPAD, D), jnp.bfloat16)]
        + [pltpu.VMEM_SHARED((RPAD, D), jnp.bfloat16)]
        + [pltpu.SemaphoreType.DMA] * 16
    ),
)
def _msg_kernel(denom_hbm, p_hbm, sd_hbm, h_hbm, acc_hbm, *scratch):
    _msg_body(denom_hbm, p_hbm, sd_hbm, h_hbm, acc_hbm, *scratch)


def _blend_body(emb_ref, acc_ref, out_ref):
    crf = (acc_ref[0].astype(jnp.float32) + acc_ref[1].astype(jnp.float32))
    out_ref[...] = (ALPHA * emb_ref[...] + BETA * crf) / (ALPHA + BETA)


def _blend(emb, acc):
    blk = 2000
    return pl.pallas_call(
        _blend_body,
        grid=(N // blk,),
        in_specs=[pl.BlockSpec((blk, D), lambda g: (g, 0)),
                  pl.BlockSpec((NC, blk, D), lambda g: (0, g, 0))],
        out_specs=pl.BlockSpec((blk, D), lambda g: (g, 0)),
        out_shape=jax.ShapeDtypeStruct((N, D), jnp.float32),
    )(emb, acc)


def kernel(embedding_input, h_input, edge_index, W_fc, W_attn):
    wa = W_attn.reshape(2, D)
    s = _scores(wa, W_fc, h_input)

    src = edge_index[0].reshape(NW, EPT)
    dst = edge_index[1].reshape(NW, EPT)
    srcp = jnp.pad(src, ((0, 0), (0, RPAD - EPT)))
    dstp = jnp.pad(dst, ((0, 0), (0, RPAD - EPT)))
    sd = jnp.concatenate([srcp.reshape(NW, ROWS, 1, 128),
                          dstp.reshape(NW, ROWS, 1, 128)], axis=2)
    sd64 = jnp.concatenate([srcp.reshape(NW, NB2, 1, BE),
                            dstp.reshape(NW, NB2, 1, BE)], axis=2)

    p, dpart = _edge_kernel(s, sd)
    denom = _dmerge(dpart).reshape(RPAD)
    hbf = jnp.pad(h_input.astype(jnp.bfloat16), ((0, RPAD - N), (0, 0)))
    acc = _msg_kernel(denom, p.reshape(NW, NB2, 1, BE), sd64, hbf)
    return _blend(embedding_input, acc)
